# baseline (speedup 1.0000x reference)
import functools

import jax
import jax.numpy as jnp
from jax import lax
from jax.experimental import pallas as pl
from jax.experimental.pallas import tpu as pltpu
from jax.experimental.pallas import tpu_sc as plsc

_N = 10000
_D = 128
_E = 320000
_DE = 16
_EPS = 1e-05

_NC = 2
_NS = 16
_NW = _NC * _NS
_C = 64
_EPW = 9984
_NCHUNK = _EPW // _C
_NEXTRA = (_E - _NW * _EPW) // _C
_RPT = 624
_RTAIL = _N - _NS * _RPT


def _edge_mm_body(a_ref, w_ref, o_ref):
    a = a_ref[...].astype(jnp.bfloat16)
    w = w_ref[...].astype(jnp.bfloat16)
    o_ref[...] = lax.dot_general(
        a, w, (((1,), (0,)), ((), ())),
        preferred_element_type=jnp.float32)


_E8 = _E // 8
_BE = 400


def _edge_mm(edge_attr8, W_big):
    return pl.pallas_call(
        _edge_mm_body,
        grid=(_E8 // _BE,),
        in_specs=[pl.BlockSpec((_BE, _D), lambda i: (i, 0)),
                  pl.BlockSpec((_D, 8 * _D), lambda i: (0, 0))],
        out_specs=pl.BlockSpec((_BE, 8 * _D), lambda i: (i, 0)),
        out_shape=jax.ShapeDtypeStruct((_E8, 8 * _D), jnp.float32),
    )(edge_attr8, W_big)


def _sc_agg(x, src, dst, e):
    mesh = plsc.VectorSubcoreMesh(core_axis_name="c", subcore_axis_name="s")

    @functools.partial(
        pl.kernel,
        mesh=mesh,
        out_type=jax.ShapeDtypeStruct((_NC * _N, _D), jnp.float32),
        scratch_types=[
            pltpu.VMEM((2, _C), jnp.int32),
            pltpu.VMEM((2, _C), jnp.int32),
            pltpu.VMEM((2, _C, _D), jnp.float32),
            pltpu.VMEM((2, _C // 8, 8 * _D), jnp.float32),
            pltpu.VMEM_SHARED((_N, _D), jnp.float32),
            pltpu.SemaphoreType.DMA,
            pltpu.SemaphoreType.DMA,
        ],
    )
    def agg_kernel(x_hbm, src_hbm, dst_hbm, e_hbm, out_hbm,
                   idxs_v, idxd_v, xrows_v, erows_v, agg_sh, sem_g, sem_e):
        cid = lax.axis_index("c")
        sid = lax.axis_index("s")
        wid = sid * _NC + cid

        zrows = xrows_v.at[0]

        @pl.loop(0, _C)
        def _(r):
            for g in range(_D // 16):
                zrows[r, pl.ds(g * 16, 16)] = jnp.zeros((16,), jnp.float32)

        zbase = sid * _RPT
        for j in range(_RPT // _C):
            pltpu.sync_copy(zrows, agg_sh.at[pl.ds(zbase + j * _C, _C)])
        _ztail = _RPT % _C
        if _ztail:
            pltpu.sync_copy(zrows.at[pl.ds(0, _ztail)],
                            agg_sh.at[pl.ds(zbase + (_RPT // _C) * _C, _ztail)])

        @pl.when(sid == _NS - 1)
        def _():
            pltpu.sync_copy(zrows.at[pl.ds(0, _RTAIL)],
                            agg_sh.at[pl.ds(_NS * _RPT, _RTAIL)])

        plsc.subcore_barrier()

        def issue(base, slot):
            ebase = pl.multiple_of(base // 8, 8)
            pltpu.sync_copy(src_hbm.at[pl.ds(base, _C)], idxs_v.at[slot])
            pltpu.sync_copy(dst_hbm.at[pl.ds(base, _C)], idxd_v.at[slot])
            pltpu.async_copy(e_hbm.at[pl.ds(ebase, _C // 8)],
                             erows_v.at[slot], sem_e)
            pltpu.async_copy(x_hbm.at[idxs_v.at[slot]], xrows_v.at[slot],
                             sem_g)

        def wait_slot(slot):
            pltpu.make_async_copy(e_hbm.at[pl.ds(0, _C // 8)],
                                  erows_v.at[slot], sem_e).wait()
            pltpu.make_async_copy(x_hbm.at[idxs_v.at[slot]],
                                  xrows_v.at[slot], sem_g).wait()

        def compute_scatter(slot):
            xr = xrows_v.at[slot]
            er = erows_v.at[slot]

            @pl.loop(0, _C // 8)
            def _(q):
                r = q * 8
                for t in range(8):
                    for g in range(_D // 16):
                        sl = pl.ds(g * 16, 16)
                        xr[r + t, sl] = jnp.maximum(
                            xr[r + t, sl] + er[q, pl.ds(t * _D + g * 16, 16)],
                            0.0)

            pltpu.sync_copy(xr, agg_sh.at[idxd_v.at[slot]], add=True)

        base0 = wid * _EPW
        issue(base0, 0)

        @pl.loop(0, _NCHUNK // 2)
        def _(i):
            g0 = 2 * i
            wait_slot(0)
            issue(base0 + (g0 + 1) * _C, 1)
            compute_scatter(0)
            wait_slot(1)

            @pl.when(g0 + 2 < _NCHUNK)
            def _():
                issue(base0 + (g0 + 2) * _C, 0)

            compute_scatter(1)

        @pl.when(wid < _NEXTRA)
        def _():
            xbase = _NW * _EPW + wid * _C
            issue(xbase, 0)
            wait_slot(0)
            compute_scatter(0)

        plsc.subcore_barrier()

        row0 = cid * _N + sid * _RPT
        pltpu.sync_copy(agg_sh.at[pl.ds(sid * _RPT, _RPT)],
                        out_hbm.at[pl.ds(row0, _RPT)])

        @pl.when(sid == _NS - 1)
        def _():
            pltpu.sync_copy(agg_sh.at[pl.ds(_NS * _RPT, _RTAIL)],
                            out_hbm.at[pl.ds(cid * _N + _NS * _RPT, _RTAIL)])

    return agg_kernel(x, src, dst, e)


def _final_body(x_ref, p0_ref, p1_ref, w_ref, b_ref, o_ref):
    h = x_ref[...] * (1.0 + _EPS) + p0_ref[...] + p1_ref[...]
    h = lax.dot_general(h, w_ref[...], (((1,), (1,)), ((), ())),
                        preferred_element_type=jnp.float32)
    o_ref[...] = jnp.maximum(h + b_ref[...], 0.0)


_BN = 2000


def _final(x, p0, p1, W_nn, b_nn2):
    return pl.pallas_call(
        _final_body,
        grid=(_N // _BN,),
        in_specs=[pl.BlockSpec((_BN, _D), lambda i: (i, 0)),
                  pl.BlockSpec((_BN, _D), lambda i: (i, 0)),
                  pl.BlockSpec((_BN, _D), lambda i: (i, 0)),
                  pl.BlockSpec((_D, _D), lambda i: (0, 0)),
                  pl.BlockSpec((1, _D), lambda i: (0, 0))],
        out_specs=pl.BlockSpec((_BN, _D), lambda i: (i, 0)),
        out_shape=jax.ShapeDtypeStruct((_N, _D), jnp.float32),
    )(x, p0, p1, W_nn, b_nn2)


def kernel(x, edge_index, edge_attr, W_edge, W_nn, b_nn):
    src = edge_index[0]
    dst = edge_index[1]
    ea8 = edge_attr.reshape(_E8, 8 * _DE)
    W_big = jnp.kron(jnp.eye(8, dtype=W_edge.dtype), W_edge.T)
    e = _edge_mm(ea8, W_big)
    parts = _sc_agg(x, src, dst, e)
    return _final(x, parts[:_N], parts[_N:], W_nn, b_nn.reshape(1, _D))


# speedup vs baseline: 1.5648x; 1.5648x over previous
import functools

import jax
import jax.numpy as jnp
from jax import lax
from jax.experimental import pallas as pl
from jax.experimental.pallas import tpu as pltpu
from jax.experimental.pallas import tpu_sc as plsc

_N = 10000
_D = 128
_E = 320000
_DE = 16
_EPS = 1e-05

_NC = 2
_NS = 16
_NW = _NC * _NS
_EPW = _E // _NW
_C = 80
_NCHUNK = _EPW // _C
_RPT = 624
_RTAIL = _N - _NS * _RPT


def _edge_mm_body(a_ref, w_ref, o_ref):
    w = w_ref[...].astype(jnp.bfloat16)
    o_ref[...] = lax.dot_general(
        a_ref[...], w, (((1,), (1,)), ((), ())),
        preferred_element_type=jnp.float32)


_BE = 2000


def _edge_mm(edge_attr_bf, W_edge):
    return pl.pallas_call(
        _edge_mm_body,
        grid=(_E // _BE,),
        in_specs=[pl.BlockSpec((_BE, _DE), lambda i: (i, 0)),
                  pl.BlockSpec((_D, _DE), lambda i: (0, 0))],
        out_specs=pl.BlockSpec((_BE, _D), lambda i: (i, 0)),
        out_shape=jax.ShapeDtypeStruct((_E, _D), jnp.float32),
    )(edge_attr_bf, W_edge)


def _sc_agg(x, src, dst, e):
    mesh = plsc.VectorSubcoreMesh(core_axis_name="c", subcore_axis_name="s")

    @functools.partial(
        pl.kernel,
        mesh=mesh,
        out_type=jax.ShapeDtypeStruct((_NC * _N, _D), jnp.float32),
        scratch_types=[
            pltpu.VMEM((2, _C), jnp.int32),
            pltpu.VMEM((2, _C), jnp.int32),
            pltpu.VMEM((2, _C, _D), jnp.float32),
            pltpu.VMEM((2, _C, _D), jnp.float32),
            pltpu.VMEM_SHARED((_N, _D), jnp.float32),
            pltpu.SemaphoreType.DMA,
            pltpu.SemaphoreType.DMA,
            pltpu.SemaphoreType.DMA,
        ],
    )
    def agg_kernel(x_hbm, src_hbm, dst_hbm, e_hbm, out_hbm,
                   idxs_v, idxd_v, xrows_v, erows_v, agg_sh,
                   sem_g, sem_e, sem_s):
        cid = lax.axis_index("c")
        sid = lax.axis_index("s")
        wid = sid * _NC + cid

        zrows = erows_v.at[0]

        @pl.loop(0, _C)
        def _(r):
            for g in range(_D // 16):
                zrows[r, pl.ds(g * 16, 16)] = jnp.zeros((16,), jnp.float32)

        zbase = sid * _RPT
        for j in range(_RPT // _C):
            pltpu.sync_copy(zrows, agg_sh.at[pl.ds(zbase + j * _C, _C)])
        _ztail = _RPT % _C
        if _ztail:
            pltpu.sync_copy(zrows.at[pl.ds(0, _ztail)],
                            agg_sh.at[pl.ds(zbase + (_RPT // _C) * _C, _ztail)])

        @pl.when(sid == _NS - 1)
        def _():
            pltpu.sync_copy(zrows.at[pl.ds(0, _RTAIL)],
                            agg_sh.at[pl.ds(_NS * _RPT, _RTAIL)])

        plsc.subcore_barrier()

        def issue(g, slot):
            base = wid * _EPW + g * _C
            pltpu.sync_copy(src_hbm.at[pl.ds(base, _C)], idxs_v.at[slot])
            pltpu.sync_copy(dst_hbm.at[pl.ds(base, _C)], idxd_v.at[slot])
            pltpu.async_copy(e_hbm.at[pl.ds(base, _C)], erows_v.at[slot],
                             sem_e)
            pltpu.async_copy(x_hbm.at[idxs_v.at[slot]], xrows_v.at[slot],
                             sem_g)

        def wait_dma(slot):
            pltpu.make_async_copy(e_hbm.at[pl.ds(0, _C)], erows_v.at[slot],
                                  sem_e).wait()
            pltpu.make_async_copy(x_hbm.at[idxs_v.at[slot]],
                                  xrows_v.at[slot], sem_g).wait()

        def compute(slot):
            xr = xrows_v.at[slot]
            er = erows_v.at[slot]

            @pl.loop(0, _C // 2)
            def _(h):
                r = h * 2
                for t in range(2):
                    for g in range(_D // 16):
                        sl = pl.ds(g * 16, 16)
                        xr[r + t, sl] = jnp.maximum(
                            xr[r + t, sl] + er[r + t, sl], 0.0)

        def scatter_async(slot):
            pltpu.async_copy(xrows_v.at[slot], agg_sh.at[idxd_v.at[slot]],
                             sem_s, add=True)

        def wait_scatter(slot):
            pltpu.make_async_copy(xrows_v.at[slot],
                                  agg_sh.at[idxd_v.at[slot]], sem_s).wait()

        issue(0, 0)
        issue(1, 1)

        @pl.loop(0, (_NCHUNK - 1) // 2)
        def _(i):
            g0 = 2 * i
            wait_dma(0)
            compute(0)
            scatter_async(0)
            wait_dma(1)
            compute(1)
            scatter_async(1)
            wait_scatter(0)
            issue(g0 + 2, 0)
            wait_scatter(1)

            @pl.when(g0 + 3 < _NCHUNK)
            def _():
                issue(g0 + 3, 1)

        wait_dma(0)
        compute(0)
        pltpu.sync_copy(xrows_v.at[0], agg_sh.at[idxd_v.at[0]], add=True)

        plsc.subcore_barrier()

        row0 = cid * _N + sid * _RPT
        pltpu.sync_copy(agg_sh.at[pl.ds(sid * _RPT, _RPT)],
                        out_hbm.at[pl.ds(row0, _RPT)])

        @pl.when(sid == _NS - 1)
        def _():
            pltpu.sync_copy(agg_sh.at[pl.ds(_NS * _RPT, _RTAIL)],
                            out_hbm.at[pl.ds(cid * _N + _NS * _RPT, _RTAIL)])

    return agg_kernel(x, src, dst, e)


def _final_body(x_ref, p0_ref, p1_ref, w_ref, b_ref, o_ref):
    h = x_ref[...] * (1.0 + _EPS) + p0_ref[...] + p1_ref[...]
    h = lax.dot_general(h, w_ref[...], (((1,), (1,)), ((), ())),
                        preferred_element_type=jnp.float32)
    o_ref[...] = jnp.maximum(h + b_ref[...], 0.0)


_BN = 2000


def _final(x, p0, p1, W_nn, b_nn2):
    return pl.pallas_call(
        _final_body,
        grid=(_N // _BN,),
        in_specs=[pl.BlockSpec((_BN, _D), lambda i: (i, 0)),
                  pl.BlockSpec((_BN, _D), lambda i: (i, 0)),
                  pl.BlockSpec((_BN, _D), lambda i: (i, 0)),
                  pl.BlockSpec((_D, _D), lambda i: (0, 0)),
                  pl.BlockSpec((1, _D), lambda i: (0, 0))],
        out_specs=pl.BlockSpec((_BN, _D), lambda i: (i, 0)),
        out_shape=jax.ShapeDtypeStruct((_N, _D), jnp.float32),
    )(x, p0, p1, W_nn, b_nn2)


def kernel(x, edge_index, edge_attr, W_edge, W_nn, b_nn):
    src = edge_index[0]
    dst = edge_index[1]
    e = _edge_mm(edge_attr.astype(jnp.bfloat16), W_edge)
    parts = _sc_agg(x, src, dst, e)
    return _final(x, parts[:_N], parts[_N:], W_nn, b_nn.reshape(1, _D))


# speedup vs baseline: 1.7378x; 1.1106x over previous
import functools

import jax
import jax.numpy as jnp
from jax import lax
from jax.experimental import pallas as pl
from jax.experimental.pallas import tpu as pltpu
from jax.experimental.pallas import tpu_sc as plsc

_N = 10000
_D = 128
_E = 320000
_DE = 16
_EPS = 1e-05

_NC = 2
_NS = 16
_NW = _NC * _NS
_EPW = _E // _NW
_C = 80
_NCHUNK = _EPW // _C
_RPT = 624
_RTAIL = _N - _NS * _RPT


def _edge_mm_body(a_ref, w_ref, o_ref):
    w = w_ref[...].astype(jnp.bfloat16)
    o_ref[...] = lax.dot_general(
        a_ref[...], w, (((1,), (1,)), ((), ())),
        preferred_element_type=jnp.float32)


_BE = 2000


def _edge_mm(edge_attr_bf, W_edge):
    return pl.pallas_call(
        _edge_mm_body,
        grid=(_E // _BE,),
        in_specs=[pl.BlockSpec((_BE, _DE), lambda i: (i, 0)),
                  pl.BlockSpec((_D, _DE), lambda i: (0, 0))],
        out_specs=pl.BlockSpec((_BE, _D), lambda i: (i, 0)),
        out_shape=jax.ShapeDtypeStruct((_E, _D), jnp.float32),
    )(edge_attr_bf, W_edge)


def _sc_agg(x, src, dst, e):
    mesh = plsc.VectorSubcoreMesh(core_axis_name="c", subcore_axis_name="s")

    @functools.partial(
        pl.kernel,
        mesh=mesh,
        out_type=jax.ShapeDtypeStruct((_NC * _N, _D), jnp.float32),
        scratch_types=[
            pltpu.VMEM((4, _C), jnp.int32),
            pltpu.VMEM((4, _C), jnp.int32),
            pltpu.VMEM((2, _C, _D), jnp.float32),
            pltpu.VMEM((2, _C, _D), jnp.float32),
            pltpu.VMEM_SHARED((_N, _D), jnp.float32),
            pltpu.SemaphoreType.DMA,
            pltpu.SemaphoreType.DMA,
            pltpu.SemaphoreType.DMA,
            pltpu.SemaphoreType.DMA,
        ],
    )
    def agg_kernel(x_hbm, src_hbm, dst_hbm, e_hbm, out_hbm,
                   idxs_v, idxd_v, xrows_v, erows_v, agg_sh,
                   sem_g, sem_e, sem_s, sem_i):
        cid = lax.axis_index("c")
        sid = lax.axis_index("s")
        wid = sid * _NC + cid

        zrows = erows_v.at[0]

        @pl.loop(0, _C)
        def _(r):
            for g in range(_D // 16):
                zrows[r, pl.ds(g * 16, 16)] = jnp.zeros((16,), jnp.float32)

        zbase = sid * _RPT
        for j in range(_RPT // _C):
            pltpu.sync_copy(zrows, agg_sh.at[pl.ds(zbase + j * _C, _C)])
        _ztail = _RPT % _C
        if _ztail:
            pltpu.sync_copy(zrows.at[pl.ds(0, _ztail)],
                            agg_sh.at[pl.ds(zbase + (_RPT // _C) * _C, _ztail)])

        @pl.when(sid == _NS - 1)
        def _():
            pltpu.sync_copy(zrows.at[pl.ds(0, _RTAIL)],
                            agg_sh.at[pl.ds(_NS * _RPT, _RTAIL)])

        plsc.subcore_barrier()

        def issue_idx(g, islot):
            base = wid * _EPW + g * _C
            pltpu.async_copy(src_hbm.at[pl.ds(base, _C)], idxs_v.at[islot],
                             sem_i)
            pltpu.async_copy(dst_hbm.at[pl.ds(base, _C)], idxd_v.at[islot],
                             sem_i)

        def wait_idx(islot):
            pltpu.make_async_copy(src_hbm.at[pl.ds(0, _C)],
                                  idxs_v.at[islot], sem_i).wait()
            pltpu.make_async_copy(dst_hbm.at[pl.ds(0, _C)],
                                  idxd_v.at[islot], sem_i).wait()

        def issue(g, slot, islot):
            base = wid * _EPW + g * _C
            pltpu.async_copy(e_hbm.at[pl.ds(base, _C)], erows_v.at[slot],
                             sem_e)
            pltpu.async_copy(x_hbm.at[idxs_v.at[islot]], xrows_v.at[slot],
                             sem_g)

        def wait_dma(slot, islot):
            pltpu.make_async_copy(e_hbm.at[pl.ds(0, _C)], erows_v.at[slot],
                                  sem_e).wait()
            pltpu.make_async_copy(x_hbm.at[idxs_v.at[islot]],
                                  xrows_v.at[slot], sem_g).wait()

        def compute(slot):
            xr = xrows_v.at[slot]
            er = erows_v.at[slot]

            @pl.loop(0, _C // 2)
            def _(h):
                r = h * 2
                for t in range(2):
                    for g in range(_D // 16):
                        sl = pl.ds(g * 16, 16)
                        xr[r + t, sl] = jnp.maximum(
                            xr[r + t, sl] + er[r + t, sl], 0.0)

        def scatter_async(slot, islot):
            pltpu.async_copy(xrows_v.at[slot], agg_sh.at[idxd_v.at[islot]],
                             sem_s, add=True)

        def wait_scatter(slot, islot):
            pltpu.make_async_copy(xrows_v.at[slot],
                                  agg_sh.at[idxd_v.at[islot]], sem_s).wait()

        issue_idx(0, 0)
        issue_idx(1, 1)
        wait_idx(0)
        wait_idx(1)
        issue(0, 0, 0)
        issue(1, 1, 1)

        @pl.loop(0, (_NCHUNK - 1) // 2)
        def _(i):
            g0 = 2 * i
            ic0 = lax.rem(g0, 4)
            ic1 = lax.rem(g0 + 1, 4)
            i2 = lax.rem(g0 + 2, 4)
            i3 = lax.rem(g0 + 3, 4)
            issue_idx(g0 + 2, i2)

            @pl.when(g0 + 3 < _NCHUNK)
            def _():
                issue_idx(g0 + 3, i3)

            wait_dma(0, ic0)
            compute(0)
            scatter_async(0, ic0)
            wait_dma(1, ic1)
            compute(1)
            scatter_async(1, ic1)
            wait_scatter(0, ic0)
            wait_idx(i2)
            issue(g0 + 2, 0, i2)
            wait_scatter(1, ic1)

            @pl.when(g0 + 3 < _NCHUNK)
            def _():
                wait_idx(i3)
                issue(g0 + 3, 1, i3)

        _ilast = (_NCHUNK - 1) % 4
        wait_dma(0, _ilast)
        compute(0)
        pltpu.sync_copy(xrows_v.at[0], agg_sh.at[idxd_v.at[_ilast]], add=True)

        plsc.subcore_barrier()

        row0 = cid * _N + sid * _RPT
        pltpu.sync_copy(agg_sh.at[pl.ds(sid * _RPT, _RPT)],
                        out_hbm.at[pl.ds(row0, _RPT)])

        @pl.when(sid == _NS - 1)
        def _():
            pltpu.sync_copy(agg_sh.at[pl.ds(_NS * _RPT, _RTAIL)],
                            out_hbm.at[pl.ds(cid * _N + _NS * _RPT, _RTAIL)])

    return agg_kernel(x, src, dst, e)


def _final_body(x_ref, p0_ref, p1_ref, w_ref, b_ref, o_ref):
    h = x_ref[...] * (1.0 + _EPS) + p0_ref[...] + p1_ref[...]
    h = lax.dot_general(h, w_ref[...], (((1,), (1,)), ((), ())),
                        preferred_element_type=jnp.float32)
    o_ref[...] = jnp.maximum(h + b_ref[...], 0.0)


_BN = 2000


def _final(x, p0, p1, W_nn, b_nn2):
    return pl.pallas_call(
        _final_body,
        grid=(_N // _BN,),
        in_specs=[pl.BlockSpec((_BN, _D), lambda i: (i, 0)),
                  pl.BlockSpec((_BN, _D), lambda i: (i, 0)),
                  pl.BlockSpec((_BN, _D), lambda i: (i, 0)),
                  pl.BlockSpec((_D, _D), lambda i: (0, 0)),
                  pl.BlockSpec((1, _D), lambda i: (0, 0))],
        out_specs=pl.BlockSpec((_BN, _D), lambda i: (i, 0)),
        out_shape=jax.ShapeDtypeStruct((_N, _D), jnp.float32),
    )(x, p0, p1, W_nn, b_nn2)


def kernel(x, edge_index, edge_attr, W_edge, W_nn, b_nn):
    src = edge_index[0]
    dst = edge_index[1]
    e = _edge_mm(edge_attr.astype(jnp.bfloat16), W_edge)
    parts = _sc_agg(x, src, dst, e)
    return _final(x, parts[:_N], parts[_N:], W_nn, b_nn.reshape(1, _D))


# speedup vs baseline: 1.7973x; 1.0342x over previous
import dataclasses
import functools

import jax
import jax.numpy as jnp
import numpy as np
from jax import lax
from jax.experimental import pallas as pl
from jax.experimental.pallas import tpu as pltpu
from jax.experimental.pallas import tpu_sc as plsc

_N = 10000
_D = 128
_E = 320000
_DE = 16
_EPS = 1e-05

_NC = 2
_NS = 16
_NW = _NC * _NS
_EPW = _E // _NW
_C = 80
_NCHUNK = _EPW // _C
_RPT = 624
_RTAIL = _N - _NS * _RPT


_LO = np.concatenate([np.arange(32 * q, 32 * q + 16) for q in range(4)])
_COLS = np.concatenate([_LO, _LO + 16])


def _edge_mm_body(a_ref, w_ref, o_ref):
    w = w_ref[...].astype(jnp.bfloat16)
    e = lax.dot_general(
        a_ref[...], w, (((1,), (0,)), ((), ())),
        preferred_element_type=jnp.float32).astype(jnp.bfloat16)
    lo = lax.bitcast_convert_type(e[:, :64], jnp.uint16).astype(jnp.uint32)
    hi = lax.bitcast_convert_type(e[:, 64:], jnp.uint16).astype(jnp.uint32)
    o_ref[...] = lax.bitcast_convert_type(lo | (hi << 16), jnp.int32)


_BE = 2000


def _edge_mm(edge_attr_bf, W_perm):
    return pl.pallas_call(
        _edge_mm_body,
        grid=(_E // _BE,),
        in_specs=[pl.BlockSpec((_BE, _DE), lambda i: (i, 0)),
                  pl.BlockSpec((_DE, _D), lambda i: (0, 0))],
        out_specs=pl.BlockSpec((_BE, _D // 2), lambda i: (i, 0)),
        out_shape=jax.ShapeDtypeStruct((_E, _D // 2), jnp.int32),
    )(edge_attr_bf, W_perm)


def _sc_cp():
    cp = pltpu.CompilerParams()
    if "needs_layout_passes" in pltpu.CompilerParams.__dataclass_fields__:
        cp = dataclasses.replace(cp, needs_layout_passes=False)
    return cp


def _sc_agg(x, src, dst, e):
    mesh = plsc.VectorSubcoreMesh(core_axis_name="c", subcore_axis_name="s")

    @functools.partial(
        pl.kernel,
        mesh=mesh,
        compiler_params=_sc_cp(),
        out_type=jax.ShapeDtypeStruct((_NC * _N, _D), jnp.float32),
        scratch_types=[
            pltpu.VMEM((4, _C), jnp.int32),
            pltpu.VMEM((4, _C), jnp.int32),
            pltpu.VMEM((2, _C, _D), jnp.float32),
            pltpu.VMEM((2, _C, _D // 2), jnp.int32),
            pltpu.VMEM_SHARED((_N, _D), jnp.float32),
            pltpu.SemaphoreType.DMA,
            pltpu.SemaphoreType.DMA,
            pltpu.SemaphoreType.DMA,
            pltpu.SemaphoreType.DMA,
        ],
    )
    def agg_kernel(x_hbm, src_hbm, dst_hbm, e_hbm, out_hbm,
                   idxs_v, idxd_v, xrows_v, erows_v, agg_sh,
                   sem_g, sem_e, sem_s, sem_i):
        cid = lax.axis_index("c")
        sid = lax.axis_index("s")
        wid = sid * _NC + cid

        zrows = xrows_v.at[0]

        @pl.loop(0, _C)
        def _(r):
            for g in range(_D // 16):
                zrows[r, pl.ds(g * 16, 16)] = jnp.zeros((16,), jnp.float32)

        zbase = sid * _RPT
        for j in range(_RPT // _C):
            pltpu.sync_copy(zrows, agg_sh.at[pl.ds(zbase + j * _C, _C)])
        _ztail = _RPT % _C
        if _ztail:
            pltpu.sync_copy(zrows.at[pl.ds(0, _ztail)],
                            agg_sh.at[pl.ds(zbase + (_RPT // _C) * _C, _ztail)])

        @pl.when(sid == _NS - 1)
        def _():
            pltpu.sync_copy(zrows.at[pl.ds(0, _RTAIL)],
                            agg_sh.at[pl.ds(_NS * _RPT, _RTAIL)])

        plsc.subcore_barrier()

        def issue_idx(g, islot):
            base = wid * _EPW + g * _C
            pltpu.async_copy(src_hbm.at[pl.ds(base, _C)], idxs_v.at[islot],
                             sem_i)
            pltpu.async_copy(dst_hbm.at[pl.ds(base, _C)], idxd_v.at[islot],
                             sem_i)

        def wait_idx(islot):
            pltpu.make_async_copy(src_hbm.at[pl.ds(0, _C)],
                                  idxs_v.at[islot], sem_i).wait()
            pltpu.make_async_copy(dst_hbm.at[pl.ds(0, _C)],
                                  idxd_v.at[islot], sem_i).wait()

        def issue(g, slot, islot):
            base = wid * _EPW + g * _C
            pltpu.async_copy(e_hbm.at[pl.ds(base, _C)], erows_v.at[slot],
                             sem_e)
            pltpu.async_copy(x_hbm.at[idxs_v.at[islot]], xrows_v.at[slot],
                             sem_g)

        def wait_dma(slot, islot):
            pltpu.make_async_copy(e_hbm.at[pl.ds(0, _C)], erows_v.at[slot],
                                  sem_e).wait()
            pltpu.make_async_copy(x_hbm.at[idxs_v.at[islot]],
                                  xrows_v.at[slot], sem_g).wait()

        def compute(slot):
            xr = xrows_v.at[slot]
            er = erows_v.at[slot]

            @pl.loop(0, _C // 2)
            def _(h):
                r = h * 2
                for t in range(2):
                    rr = r + t
                    for q in range(_D // 32):
                        ev = plsc.bitcast(er[rr, pl.ds(q * 16, 16)],
                                          jnp.bfloat16)
                        ea, eb = plsc.unpack(
                            ev, format=plsc.PackFormat.INTERLEAVED)
                        sa = pl.ds(q * 32, 16)
                        sb = pl.ds(q * 32 + 16, 16)
                        xr[rr, sa] = jnp.maximum(xr[rr, sa] + ea, 0.0)
                        xr[rr, sb] = jnp.maximum(xr[rr, sb] + eb, 0.0)

        def scatter_async(slot, islot):
            pltpu.async_copy(xrows_v.at[slot], agg_sh.at[idxd_v.at[islot]],
                             sem_s, add=True)

        def wait_scatter(slot, islot):
            pltpu.make_async_copy(xrows_v.at[slot],
                                  agg_sh.at[idxd_v.at[islot]], sem_s).wait()

        issue_idx(0, 0)
        issue_idx(1, 1)
        wait_idx(0)
        wait_idx(1)
        issue(0, 0, 0)
        issue(1, 1, 1)

        @pl.loop(0, (_NCHUNK - 1) // 2)
        def _(i):
            g0 = 2 * i
            ic0 = lax.rem(g0, 4)
            ic1 = lax.rem(g0 + 1, 4)
            i2 = lax.rem(g0 + 2, 4)
            i3 = lax.rem(g0 + 3, 4)
            issue_idx(g0 + 2, i2)

            @pl.when(g0 + 3 < _NCHUNK)
            def _():
                issue_idx(g0 + 3, i3)

            wait_dma(0, ic0)
            compute(0)
            scatter_async(0, ic0)
            wait_dma(1, ic1)
            compute(1)
            scatter_async(1, ic1)
            wait_scatter(0, ic0)
            wait_idx(i2)
            issue(g0 + 2, 0, i2)
            wait_scatter(1, ic1)

            @pl.when(g0 + 3 < _NCHUNK)
            def _():
                wait_idx(i3)
                issue(g0 + 3, 1, i3)

        _ilast = (_NCHUNK - 1) % 4
        wait_dma(0, _ilast)
        compute(0)
        pltpu.sync_copy(xrows_v.at[0], agg_sh.at[idxd_v.at[_ilast]], add=True)

        plsc.subcore_barrier()

        row0 = cid * _N + sid * _RPT
        pltpu.sync_copy(agg_sh.at[pl.ds(sid * _RPT, _RPT)],
                        out_hbm.at[pl.ds(row0, _RPT)])

        @pl.when(sid == _NS - 1)
        def _():
            pltpu.sync_copy(agg_sh.at[pl.ds(_NS * _RPT, _RTAIL)],
                            out_hbm.at[pl.ds(cid * _N + _NS * _RPT, _RTAIL)])

    return agg_kernel(x, src, dst, e)


def _final_body(x_ref, p0_ref, p1_ref, w_ref, b_ref, o_ref):
    h = x_ref[...] * (1.0 + _EPS) + p0_ref[...] + p1_ref[...]
    h = lax.dot_general(h, w_ref[...], (((1,), (1,)), ((), ())),
                        preferred_element_type=jnp.float32)
    o_ref[...] = jnp.maximum(h + b_ref[...], 0.0)


_BN = 2000


def _final(x, p0, p1, W_nn, b_nn2):
    return pl.pallas_call(
        _final_body,
        grid=(_N // _BN,),
        in_specs=[pl.BlockSpec((_BN, _D), lambda i: (i, 0)),
                  pl.BlockSpec((_BN, _D), lambda i: (i, 0)),
                  pl.BlockSpec((_BN, _D), lambda i: (i, 0)),
                  pl.BlockSpec((_D, _D), lambda i: (0, 0)),
                  pl.BlockSpec((1, _D), lambda i: (0, 0))],
        out_specs=pl.BlockSpec((_BN, _D), lambda i: (i, 0)),
        out_shape=jax.ShapeDtypeStruct((_N, _D), jnp.float32),
    )(x, p0, p1, W_nn, b_nn2)


def kernel(x, edge_index, edge_attr, W_edge, W_nn, b_nn):
    src = edge_index[0]
    dst = edge_index[1]
    W_perm = W_edge.T[:, _COLS]
    e = _edge_mm(edge_attr.astype(jnp.bfloat16), W_perm)
    parts = _sc_agg(x, src, dst, e)
    return _final(x, parts[:_N], parts[_N:], W_nn, b_nn.reshape(1, _D))


# speedup vs baseline: 2.0004x; 1.1130x over previous
import dataclasses
import functools

import jax
import jax.numpy as jnp
import numpy as np
from jax import lax
from jax.experimental import pallas as pl
from jax.experimental.pallas import tpu as pltpu
from jax.experimental.pallas import tpu_sc as plsc

_N = 10000
_D = 128
_E = 320000
_DE = 16
_EPS = 1e-05

_NC = 2
_NS = 16
_NW = _NC * _NS
_EPW = _E // _NW
_C = 80
_NCHUNK = _EPW // _C
_RPT = 624
_RTAIL = _N - _NS * _RPT


_LO = np.concatenate([np.arange(32 * q, 32 * q + 16) for q in range(4)])
_COLS = np.concatenate([_LO, _LO + 16])


def _edge_mm_body(a_ref, w_ref, o_ref):
    w = w_ref[...].astype(jnp.bfloat16)
    e = lax.dot_general(
        a_ref[...], w, (((1,), (0,)), ((), ())),
        preferred_element_type=jnp.float32).astype(jnp.bfloat16)
    lo = lax.bitcast_convert_type(e[:, :64], jnp.uint16).astype(jnp.uint32)
    hi = lax.bitcast_convert_type(e[:, 64:], jnp.uint16).astype(jnp.uint32)
    o_ref[...] = lax.bitcast_convert_type(lo | (hi << 16), jnp.int32)


_BE = 4000


def _edge_mm(edge_attr_bf, W_perm):
    return pl.pallas_call(
        _edge_mm_body,
        grid=(_E // _BE,),
        in_specs=[pl.BlockSpec((_BE, _DE), lambda i: (i, 0)),
                  pl.BlockSpec((_DE, _D), lambda i: (0, 0))],
        out_specs=pl.BlockSpec((_BE, _D // 2), lambda i: (i, 0)),
        out_shape=jax.ShapeDtypeStruct((_E, _D // 2), jnp.int32),
    )(edge_attr_bf, W_perm)


def _sc_cp():
    cp = pltpu.CompilerParams()
    if "needs_layout_passes" in pltpu.CompilerParams.__dataclass_fields__:
        cp = dataclasses.replace(cp, needs_layout_passes=False)
    return cp


def _sc_agg(x, src, dst, e):
    mesh = plsc.VectorSubcoreMesh(core_axis_name="c", subcore_axis_name="s")

    @functools.partial(
        pl.kernel,
        mesh=mesh,
        compiler_params=_sc_cp(),
        out_type=jax.ShapeDtypeStruct((_NC * _N, _D), jnp.float32),
        scratch_types=[
            pltpu.VMEM((4, _C), jnp.int32),
            pltpu.VMEM((4, _C), jnp.int32),
            pltpu.VMEM((2, _C, _D), jnp.float32),
            pltpu.VMEM((2, _C, _D // 2), jnp.int32),
            pltpu.VMEM_SHARED((_N, _D), jnp.float32),
            pltpu.SemaphoreType.DMA,
            pltpu.SemaphoreType.DMA,
            pltpu.SemaphoreType.DMA,
            pltpu.SemaphoreType.DMA,
        ],
    )
    def agg_kernel(x_hbm, src_hbm, dst_hbm, e_hbm, out_hbm,
                   idxs_v, idxd_v, xrows_v, erows_v, agg_sh,
                   sem_g, sem_e, sem_s, sem_i):
        cid = lax.axis_index("c")
        sid = lax.axis_index("s")
        wid = sid * _NC + cid

        zrows = xrows_v.at[0]

        @pl.loop(0, _C)
        def _(r):
            for g in range(_D // 16):
                zrows[r, pl.ds(g * 16, 16)] = jnp.zeros((16,), jnp.float32)

        zbase = sid * _RPT
        for j in range(_RPT // _C):
            pltpu.sync_copy(zrows, agg_sh.at[pl.ds(zbase + j * _C, _C)])
        _ztail = _RPT % _C
        if _ztail:
            pltpu.sync_copy(zrows.at[pl.ds(0, _ztail)],
                            agg_sh.at[pl.ds(zbase + (_RPT // _C) * _C, _ztail)])

        @pl.when(sid == _NS - 1)
        def _():
            pltpu.sync_copy(zrows.at[pl.ds(0, _RTAIL)],
                            agg_sh.at[pl.ds(_NS * _RPT, _RTAIL)])

        plsc.subcore_barrier()

        def issue_idx(g, islot):
            base = wid * _EPW + g * _C
            pltpu.async_copy(src_hbm.at[pl.ds(base, _C)], idxs_v.at[islot],
                             sem_i)
            pltpu.async_copy(dst_hbm.at[pl.ds(base, _C)], idxd_v.at[islot],
                             sem_i)

        def wait_idx(islot):
            pltpu.make_async_copy(src_hbm.at[pl.ds(0, _C)],
                                  idxs_v.at[islot], sem_i).wait()
            pltpu.make_async_copy(dst_hbm.at[pl.ds(0, _C)],
                                  idxd_v.at[islot], sem_i).wait()

        def issue(g, slot, islot):
            base = wid * _EPW + g * _C
            pltpu.async_copy(e_hbm.at[pl.ds(base, _C)], erows_v.at[slot],
                             sem_e)
            pltpu.async_copy(x_hbm.at[idxs_v.at[islot]], xrows_v.at[slot],
                             sem_g)

        def wait_dma(slot, islot):
            pltpu.make_async_copy(e_hbm.at[pl.ds(0, _C)], erows_v.at[slot],
                                  sem_e).wait()
            pltpu.make_async_copy(x_hbm.at[idxs_v.at[islot]],
                                  xrows_v.at[slot], sem_g).wait()

        def compute(slot):
            xr = xrows_v.at[slot]
            er = erows_v.at[slot]

            @pl.loop(0, _C // 2)
            def _(h):
                r = h * 2
                for t in range(2):
                    rr = r + t
                    for q in range(_D // 32):
                        ev = plsc.bitcast(er[rr, pl.ds(q * 16, 16)],
                                          jnp.bfloat16)
                        ea, eb = plsc.unpack(
                            ev, format=plsc.PackFormat.INTERLEAVED)
                        sa = pl.ds(q * 32, 16)
                        sb = pl.ds(q * 32 + 16, 16)
                        xr[rr, sa] = jnp.maximum(xr[rr, sa] + ea, 0.0)
                        xr[rr, sb] = jnp.maximum(xr[rr, sb] + eb, 0.0)

        def scatter_async(slot, islot):
            pltpu.async_copy(xrows_v.at[slot], agg_sh.at[idxd_v.at[islot]],
                             sem_s, add=True)

        def wait_scatter(slot, islot):
            pltpu.make_async_copy(xrows_v.at[slot],
                                  agg_sh.at[idxd_v.at[islot]], sem_s).wait()

        issue_idx(0, 0)
        issue_idx(1, 1)
        wait_idx(0)
        wait_idx(1)
        issue(0, 0, 0)
        issue(1, 1, 1)

        @pl.loop(0, (_NCHUNK - 1) // 2)
        def _(i):
            g0 = 2 * i
            ic0 = lax.rem(g0, 4)
            ic1 = lax.rem(g0 + 1, 4)
            i2 = lax.rem(g0 + 2, 4)
            i3 = lax.rem(g0 + 3, 4)
            issue_idx(g0 + 2, i2)

            @pl.when(g0 + 3 < _NCHUNK)
            def _():
                issue_idx(g0 + 3, i3)

            wait_dma(0, ic0)
            compute(0)
            scatter_async(0, ic0)
            wait_dma(1, ic1)
            compute(1)
            scatter_async(1, ic1)
            wait_scatter(0, ic0)
            wait_idx(i2)
            issue(g0 + 2, 0, i2)
            wait_scatter(1, ic1)

            @pl.when(g0 + 3 < _NCHUNK)
            def _():
                wait_idx(i3)
                issue(g0 + 3, 1, i3)

        _ilast = (_NCHUNK - 1) % 4
        wait_dma(0, _ilast)
        compute(0)
        pltpu.sync_copy(xrows_v.at[0], agg_sh.at[idxd_v.at[_ilast]], add=True)

        plsc.subcore_barrier()

        row0 = cid * _N + sid * _RPT
        pltpu.sync_copy(agg_sh.at[pl.ds(sid * _RPT, _RPT)],
                        out_hbm.at[pl.ds(row0, _RPT)])

        @pl.when(sid == _NS - 1)
        def _():
            pltpu.sync_copy(agg_sh.at[pl.ds(_NS * _RPT, _RTAIL)],
                            out_hbm.at[pl.ds(cid * _N + _NS * _RPT, _RTAIL)])

    return agg_kernel(x, src, dst, e)


def _final_body(x_ref, p0_ref, p1_ref, w_ref, b_ref, o_ref):
    h = x_ref[...] * (1.0 + _EPS) + p0_ref[...] + p1_ref[...]
    h = lax.dot_general(h, w_ref[...], (((1,), (1,)), ((), ())),
                        preferred_element_type=jnp.float32)
    o_ref[...] = jnp.maximum(h + b_ref[...], 0.0)


_BN = 2000


def _final(x, p0, p1, W_nn, b_nn2):
    return pl.pallas_call(
        _final_body,
        grid=(_N // _BN,),
        in_specs=[pl.BlockSpec((_BN, _D), lambda i: (i, 0)),
                  pl.BlockSpec((_BN, _D), lambda i: (i, 0)),
                  pl.BlockSpec((_BN, _D), lambda i: (i, 0)),
                  pl.BlockSpec((_D, _D), lambda i: (0, 0)),
                  pl.BlockSpec((1, _D), lambda i: (0, 0))],
        out_specs=pl.BlockSpec((_BN, _D), lambda i: (i, 0)),
        out_shape=jax.ShapeDtypeStruct((_N, _D), jnp.float32),
    )(x, p0, p1, W_nn, b_nn2)


def kernel(x, edge_index, edge_attr, W_edge, W_nn, b_nn):
    src = edge_index[0]
    dst = edge_index[1]
    W_perm = W_edge.T[:, _COLS]
    e = _edge_mm(edge_attr.astype(jnp.bfloat16), W_perm)
    parts = _sc_agg(x, src, dst, e)
    return _final(x, parts[:_N], parts[_N:], W_nn, b_nn.reshape(1, _D))


# speedup vs baseline: 2.1190x; 1.0593x over previous
import dataclasses
import functools

import jax
import jax.numpy as jnp
import numpy as np
from jax import lax
from jax.experimental import pallas as pl
from jax.experimental.pallas import tpu as pltpu
from jax.experimental.pallas import tpu_sc as plsc

_N = 10000
_D = 128
_E = 320000
_DE = 16
_EPS = 1e-05

_NC = 2
_NS = 16
_NW = _NC * _NS
_EPW = _E // _NW
_C = 80
_NCHUNK = _EPW // _C
_RPT = 624
_RTAIL = _N - _NS * _RPT


_LO = np.concatenate([np.arange(32 * q, 32 * q + 16) for q in range(4)])
_COLS = np.concatenate([_LO, _LO + 16])


def _edge_mm_body(a_ref, w_ref, o_ref):
    w = w_ref[...].astype(jnp.bfloat16)
    e = lax.dot_general(
        a_ref[...], w, (((1,), (0,)), ((), ())),
        preferred_element_type=jnp.float32).astype(jnp.bfloat16)
    lo = lax.bitcast_convert_type(e[:, :64], jnp.uint16).astype(jnp.uint32)
    hi = lax.bitcast_convert_type(e[:, 64:], jnp.uint16).astype(jnp.uint32)
    o_ref[...] = lax.bitcast_convert_type(lo | (hi << 16), jnp.int32)


_BE = 8000


def _edge_mm(edge_attr_bf, W_perm):
    return pl.pallas_call(
        _edge_mm_body,
        grid=(_E // _BE,),
        in_specs=[pl.BlockSpec((_BE, _DE), lambda i: (i, 0)),
                  pl.BlockSpec((_DE, _D), lambda i: (0, 0))],
        out_specs=pl.BlockSpec((_BE, _D // 2), lambda i: (i, 0)),
        out_shape=jax.ShapeDtypeStruct((_E, _D // 2), jnp.int32),
    )(edge_attr_bf, W_perm)


def _sc_cp():
    cp = pltpu.CompilerParams()
    if "needs_layout_passes" in pltpu.CompilerParams.__dataclass_fields__:
        cp = dataclasses.replace(cp, needs_layout_passes=False)
    return cp


def _sc_agg(x, src, dst, e):
    mesh = plsc.VectorSubcoreMesh(core_axis_name="c", subcore_axis_name="s")

    @functools.partial(
        pl.kernel,
        mesh=mesh,
        compiler_params=_sc_cp(),
        out_type=jax.ShapeDtypeStruct((_NC * _N, _D), jnp.float32),
        scratch_types=[
            pltpu.VMEM((4, _C), jnp.int32),
            pltpu.VMEM((4, _C), jnp.int32),
            pltpu.VMEM((2, _C, _D), jnp.float32),
            pltpu.VMEM((2, _C, _D // 2), jnp.int32),
            pltpu.VMEM_SHARED((_N, _D), jnp.float32),
            pltpu.SemaphoreType.DMA,
            pltpu.SemaphoreType.DMA,
            pltpu.SemaphoreType.DMA,
            pltpu.SemaphoreType.DMA,
        ],
    )
    def agg_kernel(x_hbm, src_hbm, dst_hbm, e_hbm, out_hbm,
                   idxs_v, idxd_v, xrows_v, erows_v, agg_sh,
                   sem_g, sem_e, sem_s, sem_i):
        cid = lax.axis_index("c")
        sid = lax.axis_index("s")
        wid = sid * _NC + cid

        zrows = xrows_v.at[0]

        @pl.loop(0, _C)
        def _(r):
            for g in range(_D // 16):
                zrows[r, pl.ds(g * 16, 16)] = jnp.zeros((16,), jnp.float32)

        zbase = sid * _RPT
        for j in range(_RPT // _C):
            pltpu.sync_copy(zrows, agg_sh.at[pl.ds(zbase + j * _C, _C)])
        _ztail = _RPT % _C
        if _ztail:
            pltpu.sync_copy(zrows.at[pl.ds(0, _ztail)],
                            agg_sh.at[pl.ds(zbase + (_RPT // _C) * _C, _ztail)])

        @pl.when(sid == _NS - 1)
        def _():
            pltpu.sync_copy(zrows.at[pl.ds(0, _RTAIL)],
                            agg_sh.at[pl.ds(_NS * _RPT, _RTAIL)])

        plsc.subcore_barrier()

        def issue_idx(g, islot):
            base = wid * _EPW + g * _C
            pltpu.async_copy(src_hbm.at[pl.ds(base, _C)], idxs_v.at[islot],
                             sem_i)
            pltpu.async_copy(dst_hbm.at[pl.ds(base, _C)], idxd_v.at[islot],
                             sem_i)

        def wait_idx(islot):
            pltpu.make_async_copy(src_hbm.at[pl.ds(0, _C)],
                                  idxs_v.at[islot], sem_i).wait()
            pltpu.make_async_copy(dst_hbm.at[pl.ds(0, _C)],
                                  idxd_v.at[islot], sem_i).wait()

        def issue(g, slot, islot):
            base = wid * _EPW + g * _C
            pltpu.async_copy(e_hbm.at[pl.ds(base, _C)], erows_v.at[slot],
                             sem_e)
            pltpu.async_copy(x_hbm.at[idxs_v.at[islot]], xrows_v.at[slot],
                             sem_g)

        def wait_dma(slot, islot):
            pltpu.make_async_copy(e_hbm.at[pl.ds(0, _C)], erows_v.at[slot],
                                  sem_e).wait()
            pltpu.make_async_copy(x_hbm.at[idxs_v.at[islot]],
                                  xrows_v.at[slot], sem_g).wait()

        def compute(slot):
            xr = xrows_v.at[slot]
            er = erows_v.at[slot]

            @pl.loop(0, _C // 2)
            def _(h):
                r = h * 2
                for t in range(2):
                    rr = r + t
                    for q in range(_D // 32):
                        ev = plsc.bitcast(er[rr, pl.ds(q * 16, 16)],
                                          jnp.bfloat16)
                        ea, eb = plsc.unpack(
                            ev, format=plsc.PackFormat.INTERLEAVED)
                        sa = pl.ds(q * 32, 16)
                        sb = pl.ds(q * 32 + 16, 16)
                        xr[rr, sa] = jnp.maximum(xr[rr, sa] + ea, 0.0)
                        xr[rr, sb] = jnp.maximum(xr[rr, sb] + eb, 0.0)

        def scatter_async(slot, islot):
            pltpu.async_copy(xrows_v.at[slot], agg_sh.at[idxd_v.at[islot]],
                             sem_s, add=True)

        def wait_scatter(slot, islot):
            pltpu.make_async_copy(xrows_v.at[slot],
                                  agg_sh.at[idxd_v.at[islot]], sem_s).wait()

        issue_idx(0, 0)
        issue_idx(1, 1)
        wait_idx(0)
        wait_idx(1)
        issue(0, 0, 0)
        issue(1, 1, 1)

        @pl.loop(0, (_NCHUNK - 1) // 2)
        def _(i):
            g0 = 2 * i
            ic0 = lax.rem(g0, 4)
            ic1 = lax.rem(g0 + 1, 4)
            i2 = lax.rem(g0 + 2, 4)
            i3 = lax.rem(g0 + 3, 4)
            issue_idx(g0 + 2, i2)

            @pl.when(g0 + 3 < _NCHUNK)
            def _():
                issue_idx(g0 + 3, i3)

            wait_dma(0, ic0)
            compute(0)
            scatter_async(0, ic0)
            wait_dma(1, ic1)
            compute(1)
            scatter_async(1, ic1)
            wait_scatter(0, ic0)
            wait_idx(i2)
            issue(g0 + 2, 0, i2)
            wait_scatter(1, ic1)

            @pl.when(g0 + 3 < _NCHUNK)
            def _():
                wait_idx(i3)
                issue(g0 + 3, 1, i3)

        _ilast = (_NCHUNK - 1) % 4
        wait_dma(0, _ilast)
        compute(0)
        pltpu.sync_copy(xrows_v.at[0], agg_sh.at[idxd_v.at[_ilast]], add=True)

        plsc.subcore_barrier()

        row0 = cid * _N + sid * _RPT
        pltpu.sync_copy(agg_sh.at[pl.ds(sid * _RPT, _RPT)],
                        out_hbm.at[pl.ds(row0, _RPT)])

        @pl.when(sid == _NS - 1)
        def _():
            pltpu.sync_copy(agg_sh.at[pl.ds(_NS * _RPT, _RTAIL)],
                            out_hbm.at[pl.ds(cid * _N + _NS * _RPT, _RTAIL)])

    return agg_kernel(x, src, dst, e)


def _final_body(x_ref, p0_ref, p1_ref, w_ref, b_ref, o_ref):
    h = x_ref[...] * (1.0 + _EPS) + p0_ref[...] + p1_ref[...]
    h = lax.dot_general(h, w_ref[...], (((1,), (1,)), ((), ())),
                        preferred_element_type=jnp.float32)
    o_ref[...] = jnp.maximum(h + b_ref[...], 0.0)


_BN = 2000


def _final(x, p0, p1, W_nn, b_nn2):
    return pl.pallas_call(
        _final_body,
        grid=(_N // _BN,),
        in_specs=[pl.BlockSpec((_BN, _D), lambda i: (i, 0)),
                  pl.BlockSpec((_BN, _D), lambda i: (i, 0)),
                  pl.BlockSpec((_BN, _D), lambda i: (i, 0)),
                  pl.BlockSpec((_D, _D), lambda i: (0, 0)),
                  pl.BlockSpec((1, _D), lambda i: (0, 0))],
        out_specs=pl.BlockSpec((_BN, _D), lambda i: (i, 0)),
        out_shape=jax.ShapeDtypeStruct((_N, _D), jnp.float32),
    )(x, p0, p1, W_nn, b_nn2)


def kernel(x, edge_index, edge_attr, W_edge, W_nn, b_nn):
    src = edge_index[0]
    dst = edge_index[1]
    W_perm = W_edge.T[:, _COLS]
    e = _edge_mm(edge_attr.astype(jnp.bfloat16), W_perm)
    parts = _sc_agg(x, src, dst, e)
    return _final(x, parts[:_N], parts[_N:], W_nn, b_nn.reshape(1, _D))


# speedup vs baseline: 2.1455x; 1.0125x over previous
import dataclasses
import functools

import jax
import jax.numpy as jnp
import numpy as np
from jax import lax
from jax.experimental import pallas as pl
from jax.experimental.pallas import tpu as pltpu
from jax.experimental.pallas import tpu_sc as plsc

_N = 10000
_D = 128
_E = 320000
_DE = 16
_EPS = 1e-05

_NC = 2
_NS = 16
_NW = _NC * _NS
_EPW = _E // _NW
_C = 80
_NCHUNK = _EPW // _C
_RPT = 624
_RTAIL = _N - _NS * _RPT


_LO = np.concatenate([np.arange(32 * q, 32 * q + 16) for q in range(4)])
_COLS = np.concatenate([_LO, _LO + 16])


def _edge_mm_body(a_ref, w_ref, o_ref):
    w = w_ref[...].astype(jnp.bfloat16)
    e = lax.dot_general(
        a_ref[...], w, (((1,), (0,)), ((), ())),
        preferred_element_type=jnp.float32).astype(jnp.bfloat16)
    lo = lax.bitcast_convert_type(e[:, :64], jnp.uint16).astype(jnp.uint32)
    hi = lax.bitcast_convert_type(e[:, 64:], jnp.uint16).astype(jnp.uint32)
    o_ref[...] = lax.bitcast_convert_type(lo | (hi << 16), jnp.int32)


_BE = 16000


def _edge_mm(edge_attr_bf, W_perm):
    return pl.pallas_call(
        _edge_mm_body,
        grid=(_E // _BE,),
        in_specs=[pl.BlockSpec((_BE, _DE), lambda i: (i, 0)),
                  pl.BlockSpec((_DE, _D), lambda i: (0, 0))],
        out_specs=pl.BlockSpec((_BE, _D // 2), lambda i: (i, 0)),
        out_shape=jax.ShapeDtypeStruct((_E, _D // 2), jnp.int32),
    )(edge_attr_bf, W_perm)


def _sc_cp():
    cp = pltpu.CompilerParams()
    if "needs_layout_passes" in pltpu.CompilerParams.__dataclass_fields__:
        cp = dataclasses.replace(cp, needs_layout_passes=False)
    return cp


def _sc_agg(x, src, dst, e):
    mesh = plsc.VectorSubcoreMesh(core_axis_name="c", subcore_axis_name="s")

    @functools.partial(
        pl.kernel,
        mesh=mesh,
        compiler_params=_sc_cp(),
        out_type=jax.ShapeDtypeStruct((_NC * _N, _D), jnp.float32),
        scratch_types=[
            pltpu.VMEM((4, _C), jnp.int32),
            pltpu.VMEM((4, _C), jnp.int32),
            pltpu.VMEM((2, _C, _D), jnp.float32),
            pltpu.VMEM((2, _C, _D // 2), jnp.int32),
            pltpu.VMEM_SHARED((_N, _D), jnp.float32),
            pltpu.SemaphoreType.DMA,
            pltpu.SemaphoreType.DMA,
            pltpu.SemaphoreType.DMA,
            pltpu.SemaphoreType.DMA,
        ],
    )
    def agg_kernel(x_hbm, src_hbm, dst_hbm, e_hbm, out_hbm,
                   idxs_v, idxd_v, xrows_v, erows_v, agg_sh,
                   sem_g, sem_e, sem_s, sem_i):
        cid = lax.axis_index("c")
        sid = lax.axis_index("s")
        wid = sid * _NC + cid

        zrows = xrows_v.at[0]

        @pl.loop(0, _C)
        def _(r):
            for g in range(_D // 16):
                zrows[r, pl.ds(g * 16, 16)] = jnp.zeros((16,), jnp.float32)

        zbase = sid * _RPT
        for j in range(_RPT // _C):
            pltpu.sync_copy(zrows, agg_sh.at[pl.ds(zbase + j * _C, _C)])
        _ztail = _RPT % _C
        if _ztail:
            pltpu.sync_copy(zrows.at[pl.ds(0, _ztail)],
                            agg_sh.at[pl.ds(zbase + (_RPT // _C) * _C, _ztail)])

        @pl.when(sid == _NS - 1)
        def _():
            pltpu.sync_copy(zrows.at[pl.ds(0, _RTAIL)],
                            agg_sh.at[pl.ds(_NS * _RPT, _RTAIL)])

        plsc.subcore_barrier()

        def issue_idx(g, islot):
            base = wid * _EPW + g * _C
            pltpu.async_copy(src_hbm.at[pl.ds(base, _C)], idxs_v.at[islot],
                             sem_i)
            pltpu.async_copy(dst_hbm.at[pl.ds(base, _C)], idxd_v.at[islot],
                             sem_i)

        def wait_idx(islot):
            pltpu.make_async_copy(src_hbm.at[pl.ds(0, _C)],
                                  idxs_v.at[islot], sem_i).wait()
            pltpu.make_async_copy(dst_hbm.at[pl.ds(0, _C)],
                                  idxd_v.at[islot], sem_i).wait()

        def issue(g, slot, islot):
            base = wid * _EPW + g * _C
            pltpu.async_copy(e_hbm.at[pl.ds(base, _C)], erows_v.at[slot],
                             sem_e)
            pltpu.async_copy(x_hbm.at[idxs_v.at[islot]], xrows_v.at[slot],
                             sem_g)

        def wait_dma(slot, islot):
            pltpu.make_async_copy(e_hbm.at[pl.ds(0, _C)], erows_v.at[slot],
                                  sem_e).wait()
            pltpu.make_async_copy(x_hbm.at[idxs_v.at[islot]],
                                  xrows_v.at[slot], sem_g).wait()

        def compute(slot):
            xr = xrows_v.at[slot]
            er = erows_v.at[slot]

            @pl.loop(0, _C // 2)
            def _(h):
                r = h * 2
                for t in range(2):
                    rr = r + t
                    for q in range(_D // 32):
                        ev = plsc.bitcast(er[rr, pl.ds(q * 16, 16)],
                                          jnp.bfloat16)
                        ea, eb = plsc.unpack(
                            ev, format=plsc.PackFormat.INTERLEAVED)
                        sa = pl.ds(q * 32, 16)
                        sb = pl.ds(q * 32 + 16, 16)
                        xr[rr, sa] = jnp.maximum(xr[rr, sa] + ea, 0.0)
                        xr[rr, sb] = jnp.maximum(xr[rr, sb] + eb, 0.0)

        def scatter_async(slot, islot):
            pltpu.async_copy(xrows_v.at[slot], agg_sh.at[idxd_v.at[islot]],
                             sem_s, add=True)

        def wait_scatter(slot, islot):
            pltpu.make_async_copy(xrows_v.at[slot],
                                  agg_sh.at[idxd_v.at[islot]], sem_s).wait()

        issue_idx(0, 0)
        issue_idx(1, 1)
        wait_idx(0)
        wait_idx(1)
        issue(0, 0, 0)
        issue(1, 1, 1)

        @pl.loop(0, (_NCHUNK - 1) // 2)
        def _(i):
            g0 = 2 * i
            ic0 = lax.rem(g0, 4)
            ic1 = lax.rem(g0 + 1, 4)
            i2 = lax.rem(g0 + 2, 4)
            i3 = lax.rem(g0 + 3, 4)
            issue_idx(g0 + 2, i2)

            @pl.when(g0 + 3 < _NCHUNK)
            def _():
                issue_idx(g0 + 3, i3)

            wait_dma(0, ic0)
            compute(0)
            scatter_async(0, ic0)
            wait_dma(1, ic1)
            compute(1)
            scatter_async(1, ic1)
            wait_scatter(0, ic0)
            wait_idx(i2)
            issue(g0 + 2, 0, i2)
            wait_scatter(1, ic1)

            @pl.when(g0 + 3 < _NCHUNK)
            def _():
                wait_idx(i3)
                issue(g0 + 3, 1, i3)

        _ilast = (_NCHUNK - 1) % 4
        wait_dma(0, _ilast)
        compute(0)
        pltpu.sync_copy(xrows_v.at[0], agg_sh.at[idxd_v.at[_ilast]], add=True)

        plsc.subcore_barrier()

        row0 = cid * _N + sid * _RPT
        pltpu.sync_copy(agg_sh.at[pl.ds(sid * _RPT, _RPT)],
                        out_hbm.at[pl.ds(row0, _RPT)])

        @pl.when(sid == _NS - 1)
        def _():
            pltpu.sync_copy(agg_sh.at[pl.ds(_NS * _RPT, _RTAIL)],
                            out_hbm.at[pl.ds(cid * _N + _NS * _RPT, _RTAIL)])

    return agg_kernel(x, src, dst, e)


def _final_body(x_ref, p0_ref, p1_ref, w_ref, b_ref, o_ref):
    h = x_ref[...] * (1.0 + _EPS) + p0_ref[...] + p1_ref[...]
    h = lax.dot_general(h, w_ref[...], (((1,), (1,)), ((), ())),
                        preferred_element_type=jnp.float32)
    o_ref[...] = jnp.maximum(h + b_ref[...], 0.0)


_BN = 2000


def _final(x, p0, p1, W_nn, b_nn2):
    return pl.pallas_call(
        _final_body,
        grid=(_N // _BN,),
        in_specs=[pl.BlockSpec((_BN, _D), lambda i: (i, 0)),
                  pl.BlockSpec((_BN, _D), lambda i: (i, 0)),
                  pl.BlockSpec((_BN, _D), lambda i: (i, 0)),
                  pl.BlockSpec((_D, _D), lambda i: (0, 0)),
                  pl.BlockSpec((1, _D), lambda i: (0, 0))],
        out_specs=pl.BlockSpec((_BN, _D), lambda i: (i, 0)),
        out_shape=jax.ShapeDtypeStruct((_N, _D), jnp.float32),
    )(x, p0, p1, W_nn, b_nn2)


def kernel(x, edge_index, edge_attr, W_edge, W_nn, b_nn):
    src = edge_index[0]
    dst = edge_index[1]
    W_perm = W_edge.T[:, _COLS]
    e = _edge_mm(edge_attr.astype(jnp.bfloat16), W_perm)
    parts = _sc_agg(x, src, dst, e)
    return _final(x, parts[:_N], parts[_N:], W_nn, b_nn.reshape(1, _D))
